# baseline (speedup 1.0000x reference)
import jax
import jax.numpy as jnp
from jax.experimental import pallas as pl
from jax.experimental.pallas import tpu as pltpu

H = 384
NE = 1024
T = 2048
B = 8
BETA = 0.25


def _dot(a, b):
    return jax.lax.dot_general(a, b, (((1,), (0,)), ((), ())),
                               preferred_element_type=jnp.float32)


def _conv3(v, wcat, bcol, out_dtype=jnp.bfloat16):
    p = _dot(wcat, v)
    p0, p1, p2 = p[0:H], p[H:2 * H], p[2 * H:3 * H]
    zero = jnp.zeros((H, 1), p.dtype)
    y = (p1 + jnp.concatenate([zero, p0[:, :-1]], axis=1)
         + jnp.concatenate([p2[:, 1:], zero], axis=1) + bcol)
    return y.astype(out_dtype)


def _vqvae_kernel(x_ref, ew1, eb1, ew2, eb2, emb, dw1, db1, dw2, db2, dw3,
                  db3, loss_out, perp_out, xhat_out, loss_acc, cnt_acc):
    b = pl.program_id(0)

    @pl.when(b == 0)
    def _():
        loss_acc[0] = 0.0
        cnt_acc[...] = jnp.zeros_like(cnt_acc)

    xb = x_ref[0].astype(jnp.bfloat16)
    h = jnp.maximum(_conv3(xb, ew1[...], eb1[...]), 0)
    z = _conv3(h, ew2[...], eb2[...])

    e = emb[...]
    s = _dot(e, z)
    ef = e.astype(jnp.float32)
    e2 = jnp.sum(ef * ef, axis=1, keepdims=True)
    c = e2 - 2.0 * s
    zf = z.astype(jnp.float32)
    z2 = jnp.sum(zf * zf, axis=0, keepdims=True)
    cmin = jnp.min(c, axis=0, keepdims=True)
    loss_acc[0] += jnp.sum(cmin + z2)
    onehot = (c == cmin).astype(jnp.bfloat16)
    ties = _dot(jnp.ones((8, NE), jnp.bfloat16), onehot)
    rtc_col = (1.0 / ties[0:1]).astype(jnp.bfloat16).reshape(T, 1)
    cnt_acc[...] += _dot(onehot, rtc_col)

    h = jnp.maximum(_conv3(z, dw1[...], db1[...]), 0)
    h = jnp.maximum(_conv3(h, dw2[...], db2[...]), 0)
    xhat_out[0] = _conv3(h, dw3[...], db3[...], jnp.float32)

    @pl.when(b == B - 1)
    def _():
        loss = (1.0 + BETA) * loss_acc[0] / (B * T * H)
        loss_out[...] = jnp.full((1, 1), loss, jnp.float32)
        em = cnt_acc[...] / (B * T)
        ent = jnp.sum(em * jnp.log(em + 1e-10))
        perp_out[...] = jnp.full((1, 1), jnp.exp(-ent), jnp.float32)


def kernel(x, enc_w1, enc_b1, enc_w2, enc_b2, embedding,
           dec_w1, dec_b1, dec_w2, dec_b2, dec_w3, dec_b3):
    tw = lambda w: jnp.transpose(w, (2, 0, 1)).reshape(3 * H, H).astype(
        jnp.bfloat16)
    col = lambda bv: bv.reshape(H, 1)

    full = lambda shp: pl.BlockSpec(shp, lambda b: (0,) * len(shp))
    wspec = full((3 * H, H))
    bspec = full((H, 1))

    loss, perp, x_hat = pl.pallas_call(
        _vqvae_kernel,
        grid=(B,),
        in_specs=[
            pl.BlockSpec((1, H, T), lambda b: (b, 0, 0)),
            wspec, bspec, wspec, bspec,
            full((NE, H)),
            wspec, bspec, wspec, bspec, wspec, bspec,
        ],
        out_specs=[
            pl.BlockSpec((1, 1), lambda b: (0, 0)),
            pl.BlockSpec((1, 1), lambda b: (0, 0)),
            pl.BlockSpec((1, H, T), lambda b: (b, 0, 0)),
        ],
        out_shape=[
            jax.ShapeDtypeStruct((1, 1), jnp.float32),
            jax.ShapeDtypeStruct((1, 1), jnp.float32),
            jax.ShapeDtypeStruct((B, H, T), jnp.float32),
        ],
        scratch_shapes=[
            pltpu.SMEM((1,), jnp.float32),
            pltpu.VMEM((NE, 1), jnp.float32),
        ],
    )(x, tw(enc_w1), col(enc_b1), tw(enc_w2), col(enc_b2),
      embedding.astype(jnp.bfloat16),
      tw(dec_w1), col(dec_b1), tw(dec_w2), col(dec_b2), tw(dec_w3),
      col(dec_b3))

    return (loss[0, 0], x_hat, perp[0, 0])


# speedup vs baseline: 1.0741x; 1.0741x over previous
import jax
import jax.numpy as jnp
from jax.experimental import pallas as pl
from jax.experimental.pallas import tpu as pltpu

H = 384
NE = 1024
T = 2048
B = 8
BETA = 0.25


def _dot(a, b):
    return jax.lax.dot_general(a, b, (((1,), (0,)), ((), ())),
                               preferred_element_type=jnp.float32)


def _conv3(v, wflat, bcol, out_dtype=jnp.bfloat16):
    zero = jnp.zeros((v.shape[0], 1), v.dtype)
    vm = jnp.concatenate([zero, v[:, :-1]], axis=1)
    vp = jnp.concatenate([v[:, 1:], zero], axis=1)
    cat = jnp.concatenate([vm, v, vp], axis=0)
    return (_dot(wflat, cat) + bcol).astype(out_dtype)


def _vqvae_kernel(x_ref, ew1, eb1, ew2, eb2, emb2, e2_ref, dw1, db1, dw2,
                  db2, dw3, db3, loss_out, perp_out, xhat_out, loss_acc,
                  cnt_acc):
    b = pl.program_id(0)

    @pl.when(b == 0)
    def _():
        loss_acc[0] = 0.0
        cnt_acc[...] = jnp.zeros_like(cnt_acc)

    xb = x_ref[0].astype(jnp.bfloat16)
    h = jnp.maximum(_conv3(xb, ew1[...], eb1[...]), 0)
    z = _conv3(h, ew2[...], eb2[...])

    s2 = _dot(emb2[...], z)
    c = e2_ref[...] + s2
    zf = z.astype(jnp.float32)
    z2 = jnp.sum(zf * zf, axis=0, keepdims=True)
    cmin = jnp.min(c, axis=0, keepdims=True)
    loss_acc[0] += jnp.sum(cmin + z2)
    onehot = (c == cmin).astype(jnp.float32)
    ties = jnp.sum(onehot, axis=0, keepdims=True)
    cnt_acc[...] += jnp.sum(onehot * (1.0 / ties), axis=1,
                            keepdims=True)

    h = jnp.maximum(_conv3(z, dw1[...], db1[...]), 0)
    h = jnp.maximum(_conv3(h, dw2[...], db2[...]), 0)
    xhat_out[0] = _conv3(h, dw3[...], db3[...], jnp.float32)

    @pl.when(b == B - 1)
    def _():
        loss = (1.0 + BETA) * loss_acc[0] / (B * T * H)
        loss_out[...] = jnp.full((1, 1), loss, jnp.float32)
        em = cnt_acc[...] / (B * T)
        ent = jnp.sum(em * jnp.log(em + 1e-10))
        perp_out[...] = jnp.full((1, 1), jnp.exp(-ent), jnp.float32)


def kernel(x, enc_w1, enc_b1, enc_w2, enc_b2, embedding,
           dec_w1, dec_b1, dec_w2, dec_b2, dec_w3, dec_b3):
    tw = lambda w: jnp.transpose(w, (0, 2, 1)).reshape(H, 3 * H).astype(
        jnp.bfloat16)
    col = lambda bv: bv.reshape(H, 1)
    emb2 = (-2.0 * embedding).astype(jnp.bfloat16)
    e2 = jnp.sum(jnp.square(embedding.astype(jnp.bfloat16)
                            .astype(jnp.float32)),
                 axis=1, keepdims=True)

    full = lambda shp: pl.BlockSpec(shp, lambda b: (0,) * len(shp))
    wspec = full((H, 3 * H))
    bspec = full((H, 1))

    loss, perp, x_hat = pl.pallas_call(
        _vqvae_kernel,
        grid=(B,),
        in_specs=[
            pl.BlockSpec((1, H, T), lambda b: (b, 0, 0)),
            wspec, bspec, wspec, bspec,
            full((NE, H)), full((NE, 1)),
            wspec, bspec, wspec, bspec, wspec, bspec,
        ],
        out_specs=[
            pl.BlockSpec((1, 1), lambda b: (0, 0)),
            pl.BlockSpec((1, 1), lambda b: (0, 0)),
            pl.BlockSpec((1, H, T), lambda b: (b, 0, 0)),
        ],
        out_shape=[
            jax.ShapeDtypeStruct((1, 1), jnp.float32),
            jax.ShapeDtypeStruct((1, 1), jnp.float32),
            jax.ShapeDtypeStruct((B, H, T), jnp.float32),
        ],
        scratch_shapes=[
            pltpu.SMEM((1,), jnp.float32),
            pltpu.VMEM((NE, 1), jnp.float32),
        ],
    )(x, tw(enc_w1), col(enc_b1), tw(enc_w2), col(enc_b2), emb2, e2,
      tw(dec_w1), col(dec_b1), tw(dec_w2), col(dec_b2), tw(dec_w3),
      col(dec_b3))

    return (loss[0, 0], x_hat, perp[0, 0])
